# baseline (speedup 1.0000x reference)
import jax
import jax.numpy as jnp
from jax.experimental import pallas as pl
from jax.experimental.pallas import tpu as pltpu

_BN_EPS = 1e-5


def _prodlda_kernel(x_ref, w_ref, o_ref, e_ref, l_ref):
    p = pl.program_id(0)
    j = pl.program_id(1)

    @pl.when(p == 0)
    def _compute():
        logits = jnp.dot(x_ref[...], w_ref[...],
                         preferred_element_type=jnp.float32)
        mu = jnp.mean(logits, axis=0, keepdims=True)
        centered = logits - mu
        var = jnp.mean(centered * centered, axis=0, keepdims=True)
        normed = centered * jax.lax.rsqrt(var + _BN_EPS)
        e = jnp.exp(normed)
        e_ref[j] = e
        s = jnp.sum(e, axis=1, keepdims=True)
        l_ref[...] = jnp.where(j == 0, s, l_ref[...] + s)

    @pl.when(p == 1)
    def _scale():
        o_ref[...] = e_ref[j] * (1.0 / l_ref[...])


def kernel(x, beta_weight_t):
    B, K = x.shape
    K2, V = beta_weight_t.shape
    assert K == K2

    block_v = 2048
    n_v = V // block_v
    assert V % block_v == 0

    xb = x.astype(jnp.bfloat16)
    wb = beta_weight_t.astype(jnp.bfloat16)

    cost = pl.CostEstimate(
        flops=2 * B * V * K,
        transcendentals=B * V,
        bytes_accessed=V * K * 2 + B * K * 2 + B * V * 4,
    )

    def x_map(p, j):
        return (0, 0)

    def w_map(p, j):
        return (0, jnp.where(p == 0, j, n_v - 1))

    def o_map(p, j):
        return (0, jnp.where(p == 0, 0, j))

    vmem_limit = int(
        B * V * 4
        + 2 * K * block_v * 2
        + B * K * 2
        + 2 * B * block_v * 4
        + (2 << 20))

    return pl.pallas_call(
        _prodlda_kernel,
        out_shape=jax.ShapeDtypeStruct((B, V), jnp.float32),
        grid=(2, n_v),
        in_specs=[
            pl.BlockSpec((B, K), x_map),
            pl.BlockSpec((K, block_v), w_map),
        ],
        out_specs=pl.BlockSpec((B, block_v), o_map),
        scratch_shapes=[
            pltpu.VMEM((n_v, B, block_v), jnp.float32),
            pltpu.VMEM((B, 1), jnp.float32),
        ],
        compiler_params=pltpu.CompilerParams(
            dimension_semantics=("arbitrary", "arbitrary"),
            vmem_limit_bytes=vmem_limit,
        ),
        cost_estimate=cost,
    )(xb, wb)
